# Initial kernel scaffold; baseline (speedup 1.0000x reference)
import jax
import jax.numpy as jnp
from jax.experimental import pallas as pl


def kernel(z, embedding_weight):
    raise NotImplementedError("write your pallas kernel here")


# speedup vs baseline: 1.3200x; 1.3200x over previous
import functools

import jax
import jax.numpy as jnp
from jax.experimental import pallas as pl

BETA = 0.25
NUM_TOKENS = 1024
CODE_DIM = 64


def _vq_kernel(z_ref, emb_ref, idx_ref, zq_ref, loss_ref):
    b = pl.program_id(0)

    zb = z_ref[0]
    emb = emb_ref[...]

    zsq_raw = jnp.sum(zb * zb, axis=0, keepdims=True)
    znorm = jnp.clip(jnp.sqrt(zsq_raw), 1e-12, None)
    zn = zb / znorm
    zsq = jnp.sum(zn * zn, axis=0, keepdims=True)

    enorm = jnp.clip(jnp.sqrt(jnp.sum(emb * emb, axis=1, keepdims=True)), 1e-12, None)
    en = emb / enorm
    en_sq = jnp.sum(en * en, axis=1, keepdims=True)

    s = jax.lax.dot_general(
        en, zn, (((1,), (0,)), ((), ())),
        preferred_element_type=jnp.float32,
    )
    d = (zsq + en_sq) - 2.0 * s
    idx = jnp.argmin(d, axis=0)

    onehot = (jax.lax.broadcasted_iota(jnp.int32, (NUM_TOKENS, 1024), 0)
              == idx[None, :]).astype(jnp.float32)
    zq = jax.lax.dot_general(
        emb, onehot, (((0,), (0,)), ((), ())),
        precision=jax.lax.Precision.HIGHEST,
        preferred_element_type=jnp.float32,
    )

    idx_ref[0, 0, :] = idx
    zq_ref[0] = zq

    diff = zq - zb
    part = jnp.sum(diff * diff).reshape(1, 1)
    loss_ref[...] = jnp.where(b == 0, part, loss_ref[...] + part)


@jax.jit
def kernel(z, embedding_weight):
    B, C, H, W = z.shape
    P = H * W
    z3 = z.reshape(B, C, P)

    idx3, zq3, loss_sum = pl.pallas_call(
        _vq_kernel,
        grid=(B,),
        in_specs=[
            pl.BlockSpec((1, C, P), lambda b: (b, 0, 0)),
            pl.BlockSpec((NUM_TOKENS, CODE_DIM), lambda b: (0, 0)),
        ],
        out_specs=[
            pl.BlockSpec((1, 1, P), lambda b: (b, 0, 0)),
            pl.BlockSpec((1, C, P), lambda b: (b, 0, 0)),
            pl.BlockSpec((1, 1), lambda b: (0, 0)),
        ],
        out_shape=[
            jax.ShapeDtypeStruct((B, 1, P), jnp.int32),
            jax.ShapeDtypeStruct((B, C, P), jnp.float32),
            jax.ShapeDtypeStruct((1, 1), jnp.float32),
        ],
    )(z3, embedding_weight)

    m = loss_sum[0, 0] / (B * C * P)
    loss = BETA * m + m
    z_q_out = zq3.reshape(B, C, H, W)
    encoding_indices = idx3.reshape(B * P)
    return (loss, z_q_out, encoding_indices)


# speedup vs baseline: 2.2997x; 1.7422x over previous
import jax
import jax.numpy as jnp
from jax.experimental import pallas as pl
from jax.experimental.pallas import tpu as pltpu

BETA = 0.25
NUM_TOKENS = 1024
CODE_DIM = 64


def _vq_kernel(z_ref, emb_ref, idx_ref, zq_ref, loss_ref, en_ref, en_sq_ref):
    b = pl.program_id(0)

    zb = z_ref[0]
    emb = emb_ref[...]

    @pl.when(b == 0)
    def _():
        enorm = jnp.clip(
            jnp.sqrt(jnp.sum(emb * emb, axis=1, keepdims=True)), 1e-12, None)
        en0 = emb / enorm
        en_ref[...] = en0
        en_sq_ref[...] = jnp.sum(en0 * en0, axis=1, keepdims=True)

    en = en_ref[...]
    en_sq = en_sq_ref[...]

    zsq_raw = jnp.sum(zb * zb, axis=0, keepdims=True)
    znorm = jnp.clip(jnp.sqrt(zsq_raw), 1e-12, None)
    zn = zb / znorm
    zsq = jnp.sum(zn * zn, axis=0, keepdims=True)

    s = jax.lax.dot_general(
        en, zn, (((1,), (0,)), ((), ())),
        preferred_element_type=jnp.float32,
    )
    d = (zsq + en_sq) - 2.0 * s
    idx = jnp.argmin(d, axis=0)

    onehot = (jax.lax.broadcasted_iota(jnp.int32, (NUM_TOKENS, 1024), 0)
              == idx[None, :]).astype(jnp.float32)
    zq = jax.lax.dot_general(
        emb, onehot, (((0,), (0,)), ((), ())),
        preferred_element_type=jnp.float32,
    )

    idx_ref[0, 0, :] = idx
    zq_ref[0] = zq

    diff = zq - zb
    part = jnp.sum(diff * diff).reshape(1, 1)
    loss_ref[...] = jnp.where(b == 0, part, loss_ref[...] + part)


@jax.jit
def kernel(z, embedding_weight):
    B, C, H, W = z.shape
    P = H * W
    z3 = z.reshape(B, C, P)

    idx3, zq3, loss_sum = pl.pallas_call(
        _vq_kernel,
        grid=(B,),
        in_specs=[
            pl.BlockSpec((1, C, P), lambda b: (b, 0, 0)),
            pl.BlockSpec((NUM_TOKENS, CODE_DIM), lambda b: (0, 0)),
        ],
        out_specs=[
            pl.BlockSpec((1, 1, P), lambda b: (b, 0, 0)),
            pl.BlockSpec((1, C, P), lambda b: (b, 0, 0)),
            pl.BlockSpec((1, 1), lambda b: (0, 0)),
        ],
        out_shape=[
            jax.ShapeDtypeStruct((B, 1, P), jnp.int32),
            jax.ShapeDtypeStruct((B, C, P), jnp.float32),
            jax.ShapeDtypeStruct((1, 1), jnp.float32),
        ],
        scratch_shapes=[
            pltpu.VMEM((NUM_TOKENS, CODE_DIM), jnp.float32),
            pltpu.VMEM((NUM_TOKENS, 1), jnp.float32),
        ],
    )(z3, embedding_weight)

    m = loss_sum[0, 0] / (B * C * P)
    loss = BETA * m + m
    z_q_out = zq3.reshape(B, C, H, W)
    encoding_indices = idx3.reshape(B * P)
    return (loss, z_q_out, encoding_indices)


# speedup vs baseline: 2.4577x; 1.0687x over previous
import jax
import jax.numpy as jnp
from jax.experimental import pallas as pl
from jax.experimental.pallas import tpu as pltpu

BETA = 0.25
NUM_TOKENS = 1024
CODE_DIM = 64
BPG = 2


def _vq_kernel(z_ref, emb_ref, idx_ref, zq_ref, loss_ref, en_ref, en_sq_ref):
    g = pl.program_id(0)
    emb = emb_ref[...]

    @pl.when(g == 0)
    def _():
        enorm = jnp.clip(
            jnp.sqrt(jnp.sum(emb * emb, axis=1, keepdims=True)), 1e-12, None)
        en0 = emb / enorm
        en_ref[...] = en0
        en_sq_ref[...] = jnp.sum(en0 * en0, axis=1, keepdims=True)

    en = en_ref[...]
    en_sq = en_sq_ref[...]

    part = jnp.zeros((1, 1), jnp.float32)
    for i in range(BPG):
        zb = z_ref[i]

        zsq_raw = jnp.sum(zb * zb, axis=0, keepdims=True)
        znorm = jnp.clip(jnp.sqrt(zsq_raw), 1e-12, None)
        zn = zb / znorm
        zsq = jnp.sum(zn * zn, axis=0, keepdims=True)

        s = jax.lax.dot_general(
            en, zn, (((1,), (0,)), ((), ())),
            preferred_element_type=jnp.float32,
        )
        d = (zsq + en_sq) - 2.0 * s
        idx = jnp.argmin(d, axis=0)

        onehot = (jax.lax.broadcasted_iota(jnp.int32, (NUM_TOKENS, 1024), 0)
                  == idx[None, :]).astype(jnp.float32)
        zq = jax.lax.dot_general(
            emb, onehot, (((0,), (0,)), ((), ())),
            preferred_element_type=jnp.float32,
        )

        idx_ref[i, 0, :] = idx
        zq_ref[i] = zq

        diff = zq - zb
        part = part + jnp.sum(diff * diff).reshape(1, 1)

    loss_ref[...] = jnp.where(g == 0, part, loss_ref[...] + part)


@jax.jit
def kernel(z, embedding_weight):
    B, C, H, W = z.shape
    P = H * W
    z3 = z.reshape(B, C, P)

    idx3, zq3, loss_sum = pl.pallas_call(
        _vq_kernel,
        grid=(B // BPG,),
        in_specs=[
            pl.BlockSpec((BPG, C, P), lambda g: (g, 0, 0)),
            pl.BlockSpec((NUM_TOKENS, CODE_DIM), lambda g: (0, 0)),
        ],
        out_specs=[
            pl.BlockSpec((BPG, 1, P), lambda g: (g, 0, 0)),
            pl.BlockSpec((BPG, C, P), lambda g: (g, 0, 0)),
            pl.BlockSpec((1, 1), lambda g: (0, 0)),
        ],
        out_shape=[
            jax.ShapeDtypeStruct((B, 1, P), jnp.int32),
            jax.ShapeDtypeStruct((B, C, P), jnp.float32),
            jax.ShapeDtypeStruct((1, 1), jnp.float32),
        ],
        scratch_shapes=[
            pltpu.VMEM((NUM_TOKENS, CODE_DIM), jnp.float32),
            pltpu.VMEM((NUM_TOKENS, 1), jnp.float32),
        ],
    )(z3, embedding_weight)

    m = loss_sum[0, 0] / (B * C * P)
    loss = BETA * m + m
    z_q_out = zq3.reshape(B, C, H, W)
    encoding_indices = idx3.reshape(B * P)
    return (loss, z_q_out, encoding_indices)


# speedup vs baseline: 2.6743x; 1.0881x over previous
import jax
import jax.numpy as jnp
from jax.experimental import pallas as pl
from jax.experimental.pallas import tpu as pltpu

BETA = 0.25
NUM_TOKENS = 1024
CODE_DIM = 64
BPG = 4


def _vq_kernel(z_ref, emb_ref, idx_ref, zq_ref, loss_ref,
               en2_ref, en_sq_ref):
    g = pl.program_id(0)
    emb = emb_ref[...]

    @pl.when(g == 0)
    def _():
        enorm = jnp.clip(
            jnp.sqrt(jnp.sum(emb * emb, axis=1, keepdims=True)), 1e-12, None)
        en0 = emb / enorm
        en2_ref[...] = -2.0 * en0
        en_sq_ref[...] = jnp.sum(en0 * en0, axis=1, keepdims=True)

    en2 = en2_ref[...]
    en_sq = en_sq_ref[...]

    part = jnp.zeros((1, 1), jnp.float32)
    for i in range(BPG):
        zb = z_ref[i]

        zsq_raw = jnp.sum(zb * zb, axis=0, keepdims=True)
        znorm = jnp.clip(jnp.sqrt(zsq_raw), 1e-12, None)
        zn = zb / znorm
        zsq = jnp.sum(zn * zn, axis=0, keepdims=True)

        s2 = jax.lax.dot_general(
            en2, zn, (((1,), (0,)), ((), ())),
            preferred_element_type=jnp.float32,
        )
        d = (zsq + en_sq) + s2
        idx = jnp.argmin(d, axis=0)

        onehot = (jax.lax.broadcasted_iota(jnp.int32, (NUM_TOKENS, 1024), 0)
                  == idx[None, :]).astype(jnp.float32)
        zq = jax.lax.dot_general(
            emb, onehot, (((0,), (0,)), ((), ())),
            preferred_element_type=jnp.float32,
        )

        idx_ref[i, 0, :] = idx
        zq_ref[i] = zq

        diff = zq - zb
        part = part + jnp.sum(diff * diff).reshape(1, 1)

    loss_ref[...] = jnp.where(g == 0, part, loss_ref[...] + part)


@jax.jit
def kernel(z, embedding_weight):
    B, C, H, W = z.shape
    P = H * W
    z3 = z.reshape(B, C, P)

    idx3, zq3, loss_sum = pl.pallas_call(
        _vq_kernel,
        grid=(B // BPG,),
        in_specs=[
            pl.BlockSpec((BPG, C, P), lambda g: (g, 0, 0)),
            pl.BlockSpec((NUM_TOKENS, CODE_DIM), lambda g: (0, 0)),
        ],
        out_specs=[
            pl.BlockSpec((BPG, 1, P), lambda g: (g, 0, 0)),
            pl.BlockSpec((BPG, C, P), lambda g: (g, 0, 0)),
            pl.BlockSpec((1, 1), lambda g: (0, 0)),
        ],
        out_shape=[
            jax.ShapeDtypeStruct((B, 1, P), jnp.int32),
            jax.ShapeDtypeStruct((B, C, P), jnp.float32),
            jax.ShapeDtypeStruct((1, 1), jnp.float32),
        ],
        scratch_shapes=[
            pltpu.VMEM((NUM_TOKENS, CODE_DIM), jnp.float32),
            pltpu.VMEM((NUM_TOKENS, 1), jnp.float32),
        ],
    )(z3, embedding_weight)

    m = loss_sum[0, 0] / (B * C * P)
    loss = BETA * m + m
    z_q_out = zq3.reshape(B, C, H, W)
    encoding_indices = idx3.reshape(B * P)
    return (loss, z_q_out, encoding_indices)
